# baseline (speedup 1.0000x reference)
import functools

import jax
import jax.numpy as jnp
from jax import lax
from jax.experimental import pallas as pl
from jax.experimental.pallas import tpu as pltpu
from jax.experimental.pallas import tpu_sc as plsc

L = 16
SCALE = 1.5
TAIL_START = 91 * 3
N_SIDE = 21 * 3


def _vgather(v, idx):
    dnums = lax.GatherDimensionNumbers(
        offset_dims=(), collapsed_slice_dims=(0,), start_index_map=(0,))
    return lax.gather(v, idx[:, None], dnums, slice_sizes=(1,),
                      mode=lax.GatherScatterMode.PROMISE_IN_BOUNDS)


def _build(F, W, chunk_frames):
    info = plsc.get_sparse_core_info()
    nc, ns = info.num_cores, info.num_subcores
    nw = nc * ns
    fpw = F // nw
    nchunk = fpw // chunk_frames
    buf_words = chunk_frames * W
    side_starts = [0, L, 2 * L, N_SIDE - L]
    phase_of = [s % 3 for s in side_starts]

    mesh = plsc.VectorSubcoreMesh(core_axis_name="c", subcore_axis_name="s")

    @functools.partial(
        pl.kernel,
        mesh=mesh,
        out_type=jax.ShapeDtypeStruct((F * W,), jnp.float32),
        scratch_types=[pltpu.VMEM((buf_words,), jnp.float32)],
    )
    def run(x_hbm, o_hbm, buf):
        wid = lax.axis_index("c") * ns + lax.axis_index("s")
        iota = lax.iota(jnp.int32, L)
        pvecs = [(iota + p) % 3 for p in range(3)]
        offs = side_starts + [N_SIDE + s for s in side_starts]

        def chunk_body(c, carry):
            f0 = wid * fpw + c * chunk_frames
            start = pl.multiple_of(f0 * W, 8)
            pltpu.sync_copy(x_hbm.at[pl.ds(start, buf_words)], buf)

            def frame_body(r, carry2):
                base = r * W + TAIL_START
                wl = buf[pl.ds(base, L)]
                wr = buf[pl.ds(base + N_SIDE, L)]
                gl = [_vgather(wl, pv) for pv in pvecs]
                gr = [_vgather(wr, pv) for pv in pvecs]
                xs = [buf[pl.ds(base + o, L)] for o in offs]
                for i, o in enumerate(offs):
                    w = (gl if i < 4 else gr)[phase_of[i % 4]]
                    buf[pl.ds(base + o, L)] = xs[i] * SCALE - w * (SCALE - 1.0)
                return carry2

            lax.fori_loop(0, chunk_frames, frame_body, 0)
            pltpu.sync_copy(buf, o_hbm.at[pl.ds(start, buf_words)])
            return carry

        lax.fori_loop(0, nchunk, chunk_body, 0)

    return run


def kernel(skeleton):
    M, T, V, C = skeleton.shape
    F, W = M * T, V * C
    run = _build(F, W, chunk_frames=120)
    out = run(skeleton.reshape(F * W))
    return out.reshape(M, T, V, C)


# speedup vs baseline: 155.3598x; 155.3598x over previous
import jax
import jax.numpy as jnp
from jax.experimental import pallas as pl

SCALE = 1.5
TAIL = 91 * 3
RSTART = 112 * 3


def _body(x_ref, w_ref, o_ref):
    j = pl.program_id(0)

    @pl.when(j < TAIL)
    def _():
        o_ref[...] = x_ref[...]

    @pl.when(j >= TAIL)
    def _():
        widx = jnp.where(j < RSTART, (j - TAIL) % 3, 3 + (j - RSTART) % 3)
        w = w_ref[pl.ds(widx, 1), :, :]
        o_ref[...] = x_ref[...] * SCALE - w * (SCALE - 1.0)


def kernel(skeleton):
    M, T, V, C = skeleton.shape
    J = V * C
    xt = jnp.transpose(skeleton, (2, 3, 1, 0)).reshape(J, T, M)
    wr = jnp.concatenate([xt[TAIL:TAIL + 3], xt[RSTART:RSTART + 3]], axis=0)
    out = pl.pallas_call(
        _body,
        grid=(J,),
        in_specs=[
            pl.BlockSpec((1, T, M), lambda j: (j, 0, 0)),
            pl.BlockSpec((6, T, M), lambda j: (0, 0, 0)),
        ],
        out_specs=pl.BlockSpec((1, T, M), lambda j: (j, 0, 0)),
        out_shape=jax.ShapeDtypeStruct((J, T, M), jnp.float32),
    )(xt, wr)
    return out.reshape(V, C, T, M).transpose(3, 2, 0, 1)


# speedup vs baseline: 448.8681x; 2.8892x over previous
import jax
import jax.numpy as jnp
from jax.experimental import pallas as pl

SCALE = 1.5
TAIL = 91 * 3
RSTART = 112 * 3


BLK = 7


def _body(x_ref, w_ref, o_ref):
    jb = pl.program_id(0)

    @pl.when(jb < TAIL // BLK)
    def _():
        o_ref[...] = x_ref[...]

    @pl.when(jb >= TAIL // BLK)
    def _():
        for p in range(BLK):
            j = jb * BLK + p
            widx = jnp.where(j < RSTART, (j - TAIL) % 3, 3 + (j - RSTART) % 3)
            w = w_ref[pl.ds(widx, 1), :, :]
            o_ref[pl.ds(p, 1), :, :] = (
                x_ref[pl.ds(p, 1), :, :] * SCALE - w * (SCALE - 1.0))


def kernel(skeleton):
    M, T, V, C = skeleton.shape
    J = V * C
    xt = jnp.transpose(skeleton, (2, 3, 1, 0)).reshape(J, T, M)
    wr = jnp.concatenate([xt[TAIL:TAIL + 3], xt[RSTART:RSTART + 3]], axis=0)
    out = pl.pallas_call(
        _body,
        grid=(J // BLK,),
        in_specs=[
            pl.BlockSpec((BLK, T, M), lambda j: (j, 0, 0)),
            pl.BlockSpec((6, T, M), lambda j: (0, 0, 0)),
        ],
        out_specs=pl.BlockSpec((BLK, T, M), lambda j: (j, 0, 0)),
        out_shape=jax.ShapeDtypeStruct((J, T, M), jnp.float32),
    )(xt, wr)
    return out.reshape(V, C, T, M).transpose(3, 2, 0, 1)


# speedup vs baseline: 460.4715x; 1.0259x over previous
import jax
import jax.numpy as jnp
from jax.experimental import pallas as pl

SCALE = 1.5
TAIL = 91 * 3
RSTART = 112 * 3
BLK = 7


def _body(x_ref, w_ref, o_ref):
    jb = pl.program_id(0)

    @pl.when(jb < TAIL // BLK)
    def _():
        o_ref[...] = x_ref[...]

    @pl.when(jb >= TAIL // BLK)
    def _():
        for p in range(BLK):
            j = jb * BLK + p
            widx = (j - jnp.where(j < RSTART, TAIL, RSTART)) % 3
            w = w_ref[pl.ds(widx, 1), :, :]
            o_ref[pl.ds(p, 1), :, :] = (
                x_ref[pl.ds(p, 1), :, :] * SCALE - w * (SCALE - 1.0))


def _wmap(j):
    return (jnp.where(j < RSTART // BLK, TAIL // BLK, RSTART // BLK), 0, 0)


def kernel(skeleton):
    M, T, V, C = skeleton.shape
    J = V * C
    xt = jnp.transpose(skeleton, (2, 3, 1, 0)).reshape(J, T, M)
    out = pl.pallas_call(
        _body,
        grid=(J // BLK,),
        in_specs=[
            pl.BlockSpec((BLK, T, M), lambda j: (j, 0, 0)),
            pl.BlockSpec((BLK, T, M), _wmap),
        ],
        out_specs=pl.BlockSpec((BLK, T, M), lambda j: (j, 0, 0)),
        out_shape=jax.ShapeDtypeStruct((J, T, M), jnp.float32),
    )(xt, xt)
    return out.reshape(V, C, T, M).transpose(3, 2, 0, 1)
